# baseline (speedup 1.0000x reference)
import functools

import jax
import jax.numpy as jnp
from jax import lax
from jax.experimental import pallas as pl
from jax.experimental.pallas import tpu as pltpu
from jax.experimental.pallas import tpu_sc as plsc

_NC, _NS = 2, 16
_NW = _NC * _NS
_CH = 128


@functools.lru_cache(maxsize=None)
def _make_gather(B, DP, dtype_name):
    dtype = jnp.dtype(dtype_name)
    b_per_w = B // _NW
    n_ch = b_per_w // _CH
    mesh = plsc.VectorSubcoreMesh(
        core_axis_name="c", subcore_axis_name="s",
        num_cores=_NC, num_subcores=_NS)
    out_sd = jax.ShapeDtypeStruct((B, DP), dtype)

    @functools.partial(
        pl.kernel,
        out_type=(out_sd, out_sd),
        mesh=mesh,
        scratch_types=[
            pltpu.VMEM((2, n_ch, _CH), jnp.int32),
            pltpu.VMEM((2, _CH, DP), dtype),
            pltpu.VMEM((2, _CH, DP), dtype),
            pltpu.SemaphoreType.DMA,
            pltpu.SemaphoreType.DMA,
        ],
        compiler_params=pltpu.CompilerParams(use_tc_tiling_on_sc=True),
    )
    def gather_kernel(idx_hbm, users_hbm, items_hbm, u_out, i_out,
                      idx_v, ubuf_v, ibuf_v, usem, isem):
        wid = lax.axis_index("s") * _NC + lax.axis_index("c")
        base = wid * b_per_w
        pltpu.sync_copy(idx_hbm.at[wid], idx_v)
        tables = (
            (users_hbm, u_out, ubuf_v, usem, 0),
            (items_hbm, i_out, ibuf_v, isem, 1),
        )
        copies = {0: [None, None], 1: [None, None]}
        for j in range(n_ch):
            b = j % 2
            for tbl, out, buf, sem, t in tables:
                if copies[t][b] is not None:
                    copies[t][b].wait()
                    pltpu.sync_copy(
                        buf.at[b],
                        out.at[pl.ds(base + (j - 2) * _CH, _CH)])
                copies[t][b] = pltpu.async_copy(
                    tbl.at[idx_v.at[t, j]], buf.at[b], sem)
        for j in range(max(n_ch - 2, 0), n_ch):
            b = j % 2
            for tbl, out, buf, sem, t in tables:
                copies[t][b].wait()
                pltpu.sync_copy(
                    buf.at[b], out.at[pl.ds(base + j * _CH, _CH)])

    return gather_kernel


def kernel(inputs, users_table, items_table):
    B = inputs.shape[0]
    V, D = users_table.shape
    b_per_w = B // _NW
    n_ch = b_per_w // _CH
    users_p = users_table.reshape(V // 2, 2 * D)
    items_p = items_table.reshape(V // 2, 2 * D)
    sidx = (inputs >> 1).T.reshape(2, _NW, n_ch, _CH).transpose(1, 0, 2, 3)
    f = _make_gather(B, 2 * D, str(users_table.dtype))
    u128, i128 = f(sidx, users_p, items_p)
    half = inputs & 1
    u = jnp.where(half[:, 0:1] == 0, u128[:, :D], u128[:, D:])
    i = jnp.where(half[:, 1:2] == 0, i128[:, :D], i128[:, D:])
    return (u, i)


# speedup vs baseline: 1.1439x; 1.1439x over previous
import functools

import jax
import jax.numpy as jnp
from jax import lax
from jax.experimental import pallas as pl
from jax.experimental.pallas import tpu as pltpu
from jax.experimental.pallas import tpu_sc as plsc

_NC, _NS = 2, 16
_NW = _NC * _NS
_CH = 128
_BR = 1024


def _transpose_block(lo_ref, hi_ref, o_ref):
    o_ref[...] = jnp.concatenate([lo_ref[...].T, hi_ref[...].T], axis=1)


@functools.lru_cache(maxsize=None)
def _make_repack(C, R, dtype_name):
    dtype = jnp.dtype(dtype_name)
    grid = (R + 2 * _BR - 1) // (2 * _BR)
    return pl.pallas_call(
        _transpose_block,
        grid=(grid,),
        in_specs=[
            pl.BlockSpec((C, _BR), lambda k: (0, 2 * k)),
            pl.BlockSpec((C, _BR), lambda k: (0, 2 * k + 1)),
        ],
        out_specs=pl.BlockSpec((_BR, 2 * C), lambda k: (k, 0)),
        out_shape=jax.ShapeDtypeStruct((grid * _BR, 2 * C), dtype),
    )


@functools.lru_cache(maxsize=None)
def _make_gather(B, DP, dtype_name):
    dtype = jnp.dtype(dtype_name)
    b_per_w = B // _NW
    n_ch = b_per_w // _CH
    mesh = plsc.VectorSubcoreMesh(
        core_axis_name="c", subcore_axis_name="s",
        num_cores=_NC, num_subcores=_NS)

    @functools.partial(
        pl.kernel,
        out_type=jax.ShapeDtypeStruct((B, DP), dtype),
        mesh=mesh,
        scratch_types=[
            pltpu.VMEM((n_ch, _CH), jnp.int32),
            pltpu.VMEM((2, _CH, DP), dtype),
            pltpu.SemaphoreType.DMA,
        ],
        compiler_params=pltpu.CompilerParams(use_tc_tiling_on_sc=True),
    )
    def gather_kernel(idx_hbm, tbl_hbm, out_hbm, idx_v, buf_v, sem):
        wid = lax.axis_index("s") * _NC + lax.axis_index("c")
        base = wid * b_per_w
        pltpu.sync_copy(idx_hbm.at[wid], idx_v)
        copies = [None, None]
        for j in range(n_ch):
            b = j % 2
            if copies[b] is not None:
                copies[b].wait()
                pltpu.sync_copy(
                    buf_v.at[b],
                    out_hbm.at[pl.ds(base + (j - 2) * _CH, _CH)])
            copies[b] = pltpu.async_copy(
                tbl_hbm.at[idx_v.at[j]], buf_v.at[b], sem)
        for j in range(max(n_ch - 2, 0), n_ch):
            b = j % 2
            copies[b].wait()
            pltpu.sync_copy(
                buf_v.at[b], out_hbm.at[pl.ds(base + j * _CH, _CH)])

    return gather_kernel


def kernel(inputs, users_table, items_table):
    B = inputs.shape[0]
    V, D = users_table.shape
    b_per_w = B // _NW
    n_ch = b_per_w // _CH
    dt = str(users_table.dtype)
    repack = _make_repack(D, V, dt)
    gather = _make_gather(B, 2 * D, dt)
    ut, it_ = users_table.T, items_table.T
    up = repack(ut, ut)
    ip = repack(it_, it_)
    r_u, r_i = inputs[:, 0], inputs[:, 1]
    sidx_u = (((r_u >> 11) << 10) | (r_u & 1023)).reshape(_NW, n_ch, _CH)
    sidx_i = (((r_i >> 11) << 10) | (r_i & 1023)).reshape(_NW, n_ch, _CH)
    u128 = gather(sidx_u, up)
    i128 = gather(sidx_i, ip)
    u = jnp.where((r_u & 1024)[:, None] != 0, u128[:, D:], u128[:, :D])
    i = jnp.where((r_i & 1024)[:, None] != 0, i128[:, D:], i128[:, :D])
    return (u, i)


# speedup vs baseline: 1.2163x; 1.0633x over previous
import functools

import jax
import jax.numpy as jnp
from jax import lax
from jax.experimental import pallas as pl
from jax.experimental.pallas import tpu as pltpu
from jax.experimental.pallas import tpu_sc as plsc

_NC, _NS = 2, 16
_NW = _NC * _NS
_CH = 128
_BR = 1024


def _transpose_block(lo_ref, hi_ref, o_ref):
    z = jnp.concatenate([lo_ref[...], hi_ref[...]], axis=0)
    ident = jnp.eye(z.shape[0], dtype=z.dtype)
    o_ref[...] = jax.lax.dot_general(
        z, ident, (((0,), (0,)), ((), ())),
        preferred_element_type=z.dtype)


@functools.lru_cache(maxsize=None)
def _make_repack(C, R, dtype_name):
    dtype = jnp.dtype(dtype_name)
    grid = (R + 2 * _BR - 1) // (2 * _BR)
    return pl.pallas_call(
        _transpose_block,
        grid=(grid,),
        in_specs=[
            pl.BlockSpec((C, _BR), lambda k: (0, 2 * k)),
            pl.BlockSpec((C, _BR), lambda k: (0, 2 * k + 1)),
        ],
        out_specs=pl.BlockSpec((_BR, 2 * C), lambda k: (k, 0)),
        out_shape=jax.ShapeDtypeStruct((grid * _BR, 2 * C), dtype),
    )


@functools.lru_cache(maxsize=None)
def _make_gather(B, DP, dtype_name):
    dtype = jnp.dtype(dtype_name)
    b_per_w = B // _NW
    n_ch = b_per_w // _CH
    mesh = plsc.VectorSubcoreMesh(
        core_axis_name="c", subcore_axis_name="s",
        num_cores=_NC, num_subcores=_NS)

    @functools.partial(
        pl.kernel,
        out_type=jax.ShapeDtypeStruct((B, DP), dtype),
        mesh=mesh,
        scratch_types=[
            pltpu.VMEM((n_ch, _CH), jnp.int32),
            pltpu.VMEM((2, _CH, DP), dtype),
            pltpu.SemaphoreType.DMA,
        ],
        compiler_params=pltpu.CompilerParams(use_tc_tiling_on_sc=True),
    )
    def gather_kernel(idx_hbm, tbl_hbm, out_hbm, idx_v, buf_v, sem):
        wid = lax.axis_index("s") * _NC + lax.axis_index("c")
        base = wid * b_per_w
        pltpu.sync_copy(idx_hbm.at[wid], idx_v)
        copies = [None, None]
        for j in range(n_ch):
            b = j % 2
            if copies[b] is not None:
                copies[b].wait()
                pltpu.sync_copy(
                    buf_v.at[b],
                    out_hbm.at[pl.ds(base + (j - 2) * _CH, _CH)])
            copies[b] = pltpu.async_copy(
                tbl_hbm.at[idx_v.at[j]], buf_v.at[b], sem)
        for j in range(max(n_ch - 2, 0), n_ch):
            b = j % 2
            copies[b].wait()
            pltpu.sync_copy(
                buf_v.at[b], out_hbm.at[pl.ds(base + j * _CH, _CH)])

    return gather_kernel


def kernel(inputs, users_table, items_table):
    B = inputs.shape[0]
    V, D = users_table.shape
    b_per_w = B // _NW
    n_ch = b_per_w // _CH
    dt = str(users_table.dtype)
    repack = _make_repack(D, V, dt)
    gather = _make_gather(B, 2 * D, dt)
    ut, it_ = users_table.T, items_table.T
    up = repack(ut, ut)
    ip = repack(it_, it_)
    r_u, r_i = inputs[:, 0], inputs[:, 1]
    sidx_u = (((r_u >> 11) << 10) | (r_u & 1023)).reshape(_NW, n_ch, _CH)
    sidx_i = (((r_i >> 11) << 10) | (r_i & 1023)).reshape(_NW, n_ch, _CH)
    u128 = gather(sidx_u, up)
    i128 = gather(sidx_i, ip)
    u = jnp.where((r_u & 1024)[:, None] != 0, u128[:, D:], u128[:, :D])
    i = jnp.where((r_i & 1024)[:, None] != 0, i128[:, D:], i128[:, :D])
    return (u, i)
